# Initial kernel scaffold; baseline (speedup 1.0000x reference)
import jax
import jax.numpy as jnp
from jax.experimental import pallas as pl


def kernel(frac_app_idx, all_true_idx, all_false_idx, frac_tf_idx, frac_app_tab, true_tab, false_tab, frac_tf_tab, W1, b1, W2, b2):
    raise NotImplementedError("write your pallas kernel here")


# speedup vs baseline: 229.7956x; 229.7956x over previous
import functools

import jax
import jax.numpy as jnp
import numpy as np
from jax.experimental import pallas as pl
from jax.experimental.pallas import tpu as pltpu

L = 160
DIN = L * 8
BB = 1024


def _fused_kernel(fa_ref, tt_ref, ff_ref, ft_ref,
                  fa_tab_ref, tt_tab_ref, ff_tab_ref, ft_tab_ref,
                  w1_ref, b1_ref, w2_ref, b2_ref,
                  out_ref, feat_ref):
    def lookup11(idx_ref, tab_ref, base):
        idx = idx_ref[...]
        acc0 = jnp.full(idx.shape, tab_ref[0, 0], jnp.float32)
        acc1 = jnp.full(idx.shape, tab_ref[0, 1], jnp.float32)
        for k in range(1, 11):
            m = idx == k
            acc0 = jnp.where(m, tab_ref[k, 0], acc0)
            acc1 = jnp.where(m, tab_ref[k, 1], acc1)
        feat_ref[:, base:base + L] = acc0.astype(jnp.bfloat16)
        feat_ref[:, base + L:base + 2 * L] = acc1.astype(jnp.bfloat16)

    def lookup2(idx_ref, tab_ref, base):
        f = idx_ref[...].astype(jnp.float32)
        v0 = tab_ref[0, 0] + f * (tab_ref[1, 0] - tab_ref[0, 0])
        v1 = tab_ref[0, 1] + f * (tab_ref[1, 1] - tab_ref[0, 1])
        feat_ref[:, base:base + L] = v0.astype(jnp.bfloat16)
        feat_ref[:, base + L:base + 2 * L] = v1.astype(jnp.bfloat16)

    lookup11(fa_ref, fa_tab_ref, 0)
    lookup2(tt_ref, tt_tab_ref, 2 * L)
    lookup2(ff_ref, ff_tab_ref, 4 * L)
    lookup11(ft_ref, ft_tab_ref, 6 * L)

    feat = feat_ref[...]
    h = jnp.dot(feat, w1_ref[...], preferred_element_type=jnp.float32)
    h = jnp.maximum(h + b1_ref[...], 0.0).astype(jnp.bfloat16)
    out = jnp.dot(h, w2_ref[...], preferred_element_type=jnp.float32)
    out_ref[...] = out + b2_ref[...]


@functools.partial(jax.jit, static_argnames=())
def kernel(frac_app_idx, all_true_idx, all_false_idx, frac_tf_idx,
           frac_app_tab, true_tab, false_tab, frac_tf_tab,
           W1, b1, W2, b2):
    B = frac_app_idx.shape[0]
    H2 = W1.shape[1]
    H = W2.shape[1]
    bb = min(BB, B)

    perm = np.array([t * 2 * L + 2 * l + c
                     for t in range(4) for c in range(2) for l in range(L)],
                    dtype=np.int32)
    W1p = jnp.take(W1, jnp.asarray(perm), axis=0).astype(jnp.bfloat16)
    W2b = W2.astype(jnp.bfloat16)

    smem = pl.BlockSpec(memory_space=pltpu.SMEM)
    grid = (B // bb,)
    out = pl.pallas_call(
        _fused_kernel,
        grid=grid,
        in_specs=[
            pl.BlockSpec((bb, L), lambda i: (i, 0)),
            pl.BlockSpec((bb, L), lambda i: (i, 0)),
            pl.BlockSpec((bb, L), lambda i: (i, 0)),
            pl.BlockSpec((bb, L), lambda i: (i, 0)),
            smem, smem, smem, smem,
            pl.BlockSpec((DIN, H2), lambda i: (0, 0)),
            pl.BlockSpec((1, H2), lambda i: (0, 0)),
            pl.BlockSpec((H2, H), lambda i: (0, 0)),
            pl.BlockSpec((1, H), lambda i: (0, 0)),
        ],
        out_specs=pl.BlockSpec((bb, H), lambda i: (i, 0)),
        out_shape=jax.ShapeDtypeStruct((B, H), jnp.float32),
        scratch_shapes=[pltpu.VMEM((bb, DIN), jnp.bfloat16)],
    )(frac_app_idx, all_true_idx, all_false_idx, frac_tf_idx,
      frac_app_tab, true_tab, false_tab, frac_tf_tab,
      W1p, b1.reshape(1, H2), W2b, b2.reshape(1, H))
    return out


# speedup vs baseline: 249.9166x; 1.0876x over previous
import functools

import jax
import jax.numpy as jnp
import numpy as np
from jax.experimental import pallas as pl
from jax.experimental.pallas import tpu as pltpu

L = 160
DIN = L * 8
BB = 1024


def _fused_kernel(fa_ref, tt_ref, ff_ref, ft_ref,
                  fa_tab_ref, tt_tab_ref, ff_tab_ref, ft_tab_ref,
                  w1_ref, b1_ref, w2_ref, b2_ref,
                  out_ref, feat_ref):
    def lookup11(idx_ref, tab_ref, base):
        idx = idx_ref[...].astype(jnp.bfloat16)
        acc0 = jnp.full(idx.shape, tab_ref[0, 0], jnp.bfloat16)
        acc1 = jnp.full(idx.shape, tab_ref[0, 1], jnp.bfloat16)
        for k in range(1, 11):
            m = idx == k
            acc0 = jnp.where(m, jnp.bfloat16(tab_ref[k, 0]), acc0)
            acc1 = jnp.where(m, jnp.bfloat16(tab_ref[k, 1]), acc1)
        feat_ref[:, base:base + L] = acc0
        feat_ref[:, base + L:base + 2 * L] = acc1

    def lookup2(idx_ref, tab_ref, base):
        m = idx_ref[...].astype(jnp.bfloat16) == 1
        feat_ref[:, base:base + L] = jnp.where(
            m, jnp.bfloat16(tab_ref[1, 0]), jnp.bfloat16(tab_ref[0, 0]))
        feat_ref[:, base + L:base + 2 * L] = jnp.where(
            m, jnp.bfloat16(tab_ref[1, 1]), jnp.bfloat16(tab_ref[0, 1]))

    lookup11(fa_ref, fa_tab_ref, 0)
    lookup2(tt_ref, tt_tab_ref, 2 * L)
    lookup2(ff_ref, ff_tab_ref, 4 * L)
    lookup11(ft_ref, ft_tab_ref, 6 * L)

    feat = feat_ref[...]
    h = jnp.dot(feat, w1_ref[...], preferred_element_type=jnp.float32)
    h = jnp.maximum(h + b1_ref[...], 0.0).astype(jnp.bfloat16)
    out = jnp.dot(h, w2_ref[...], preferred_element_type=jnp.float32)
    out_ref[...] = out + b2_ref[...]


@functools.partial(jax.jit, static_argnames=())
def kernel(frac_app_idx, all_true_idx, all_false_idx, frac_tf_idx,
           frac_app_tab, true_tab, false_tab, frac_tf_tab,
           W1, b1, W2, b2):
    B = frac_app_idx.shape[0]
    H2 = W1.shape[1]
    H = W2.shape[1]
    bb = min(BB, B)

    W1p = (W1.reshape(4, L, 2, H2).transpose(0, 2, 1, 3)
           .reshape(DIN, H2).astype(jnp.bfloat16))
    W2b = W2.astype(jnp.bfloat16)

    smem = pl.BlockSpec(memory_space=pltpu.SMEM)
    grid = (B // bb,)
    out = pl.pallas_call(
        _fused_kernel,
        grid=grid,
        in_specs=[
            pl.BlockSpec((bb, L), lambda i: (i, 0)),
            pl.BlockSpec((bb, L), lambda i: (i, 0)),
            pl.BlockSpec((bb, L), lambda i: (i, 0)),
            pl.BlockSpec((bb, L), lambda i: (i, 0)),
            smem, smem, smem, smem,
            pl.BlockSpec((DIN, H2), lambda i: (0, 0)),
            pl.BlockSpec((1, H2), lambda i: (0, 0)),
            pl.BlockSpec((H2, H), lambda i: (0, 0)),
            pl.BlockSpec((1, H), lambda i: (0, 0)),
        ],
        out_specs=pl.BlockSpec((bb, H), lambda i: (i, 0)),
        out_shape=jax.ShapeDtypeStruct((B, H), jnp.float32),
        scratch_shapes=[pltpu.VMEM((bb, DIN), jnp.bfloat16)],
    )(frac_app_idx, all_true_idx, all_false_idx, frac_tf_idx,
      frac_app_tab, true_tab, false_tab, frac_tf_tab,
      W1p, b1.reshape(1, H2), W2b, b2.reshape(1, H))
    return out
